# baseline (speedup 1.0000x reference)
import functools
import math

import jax
import jax.numpy as jnp
from jax import lax
from jax.experimental import pallas as pl
from jax.experimental.pallas import tpu as pltpu
from jax.experimental.pallas import tpu_sc as plsc

N = 10000
C = 128
G = 64
G3 = G * G * G
K27 = 27
EPS = 1e-4

NSC = 2
NT = 16
NW = NSC * NT
BN = 320
NP = NW * BN
D2 = NT * BN
TRASH = D2
ACC_ROWS = D2 + 8
GCH = G3 // NW
NB16 = (BN + 15) // 16
NSLOT = K27 * NB16
CH = 128
KMAX = (NSLOT * 16 + CH - 1) // CH

_MESH = plsc.VectorSubcoreMesh(core_axis_name="c", subcore_axis_name="s")
_SC_PARAMS = pltpu.CompilerParams(needs_layout_passes=False)


def _iota16():
    return lax.iota(jnp.int32, 16)


@functools.partial(
    pl.kernel,
    out_type=jax.ShapeDtypeStruct((G3,), jnp.int32),
    compiler_params=_SC_PARAMS,
    mesh=_MESH,
    scratch_types=[
        pltpu.VMEM((N * 3,), jnp.float32),
        pltpu.VMEM((GCH,), jnp.int32),
    ],
)
def _grid_kernel(pos_hbm, grid_hbm, posv, gchunk):
    c = lax.axis_index("c")
    s = lax.axis_index("s")
    w = c * NT + s
    base = w * GCH
    pltpu.sync_copy(pos_hbm, posv)

    def fill(i, _):
        gchunk[pl.ds(i * 16, 16)] = jnp.full((16,), -1, jnp.int32)
        return 0

    lax.fori_loop(0, GCH // 16, fill, 0)

    def scan(i, _):
        n = i * 16 + _iota16()
        f = n * 3
        xi = plsc.load_gather(posv, [f]).astype(jnp.int32)
        yi = plsc.load_gather(posv, [f + 1]).astype(jnp.int32)
        zi = plsc.load_gather(posv, [f + 2]).astype(jnp.int32)
        key = xi * (G * G) + yi * G + zi
        m = (key >= base) & (key < base + GCH)
        plsc.store_scatter(gchunk, [key - base], n, mask=m)
        return 0

    lax.fori_loop(0, N // 16, scan, 0)
    pltpu.sync_copy(gchunk, grid_hbm.at[pl.ds(base, GCH)])


@functools.partial(
    pl.kernel,
    out_type=(
        jax.ShapeDtypeStruct((NW, KMAX, CH), jnp.int32),
        jax.ShapeDtypeStruct((NW, KMAX, CH), jnp.int32),
    ),
    compiler_params=_SC_PARAMS,
    mesh=_MESH,
    scratch_types=[
        pltpu.VMEM((N * 3,), jnp.float32),
        pltpu.VMEM((KMAX, CH), jnp.int32),
        pltpu.VMEM((KMAX, CH), jnp.int32),
        pltpu.VMEM((KMAX, CH), jnp.int32),
        pltpu.VMEM((KMAX, CH), jnp.int32),
        pltpu.VMEM((KMAX, CH), jnp.int32),
        pltpu.SemaphoreType.DMA,
    ],
)
def _edge_kernel(pos_hbm, grid_hbm, ridx_hbm, dstl_hbm,
                 posv, nkv, inbv, gv, rv, dv, sem):
    c = lax.axis_index("c")
    s = lax.axis_index("s")
    w = c * NT + s
    base_n = w * BN
    pltpu.sync_copy(pos_hbm, posv)

    def kbody(k, _):
        dx = k // 9 - 1
        dy = (k // 3) % 3 - 1
        dz = k % 3 - 1

        def ibody(i, _):
            ln = i * 16 + _iota16()
            n = base_n + ln
            f = jnp.minimum(n, N - 1) * 3
            xi = plsc.load_gather(posv, [f]).astype(jnp.int32)
            yi = plsc.load_gather(posv, [f + 1]).astype(jnp.int32)
            zi = plsc.load_gather(posv, [f + 2]).astype(jnp.int32)
            nx = xi + dx
            ny = yi + dy
            nz = zi + dz
            inb = ((nx >= 0) & (nx < G) & (ny >= 0) & (ny < G)
                   & (nz >= 0) & (nz < G) & (n < N))
            nkey = nx * (G * G) + ny * G + nz
            j = k * NB16 + i
            row = j // 8
            col = (j % 8) * 16
            nkv[row, pl.ds(col, 16)] = jnp.where(inb, nkey, 0)
            inbv[row, pl.ds(col, 16)] = inb.astype(jnp.int32)
            return 0

        lax.fori_loop(0, NB16, ibody, 0)
        return 0

    lax.fori_loop(0, K27, kbody, 0)

    for t in range(8 - (NSLOT % 8)):
        col = (NSLOT % 8 + t) * 16
        rv[KMAX - 1, pl.ds(col, 16)] = jnp.zeros((16,), jnp.int32)
        dv[KMAX - 1, pl.ds(col, 16)] = jnp.full((16,), TRASH, jnp.int32)

    def fire(j, _):
        pltpu.async_copy(grid_hbm.at[nkv.at[j]], gv.at[j], sem)
        return 0

    lax.fori_loop(0, KMAX, fire, 0)
    pltpu.make_async_copy(ridx_hbm.at[w], gv, sem).wait()

    def kbody2(k, _):
        def ibody2(i, _):
            j = k * NB16 + i
            row = j // 8
            col = (j % 8) * 16
            g = gv[row, pl.ds(col, 16)]
            inb = inbv[row, pl.ds(col, 16)]
            valid = (inb > 0) & (g >= 0)
            ln = i * 16 + _iota16()
            n = base_n + ln
            rv[row, pl.ds(col, 16)] = jnp.where(valid, g * K27 + k, 0)
            dv[row, pl.ds(col, 16)] = jnp.where(valid, n - c * D2, TRASH)
            return 0

        lax.fori_loop(0, NB16, ibody2, 0)
        return 0

    lax.fori_loop(0, K27, kbody2, 0)
    pltpu.sync_copy(rv, ridx_hbm.at[w])
    pltpu.sync_copy(dv, dstl_hbm.at[w])


@functools.partial(
    pl.kernel,
    out_type=jax.ShapeDtypeStruct((NP, C), jnp.float32),
    compiler_params=_SC_PARAMS,
    mesh=_MESH,
    scratch_types=[
        pltpu.VMEM((KMAX, CH), jnp.int32),
        pltpu.VMEM((KMAX, CH), jnp.int32),
        pltpu.VMEM((CH, C), jnp.float32),
        pltpu.VMEM_SHARED((ACC_ROWS, C), jnp.float32),
        pltpu.SemaphoreType.DMA,
    ],
)
def _acc_kernel(y_hbm, init_hbm, ridx_hbm, dstl_hbm, out_hbm,
                rixv, dixv, gbuf, accum, sem):
    c = lax.axis_index("c")
    s = lax.axis_index("s")
    w = c * NT + s
    pltpu.sync_copy(ridx_hbm.at[w], rixv)
    pltpu.sync_copy(dstl_hbm.at[w], dixv)
    pltpu.sync_copy(init_hbm.at[pl.ds(w * BN, BN)],
                    accum.at[pl.ds(s * BN, BN)])

    def chunk(j, _):
        pltpu.async_copy(y_hbm.at[rixv.at[j]], gbuf, sem).wait()
        pltpu.sync_copy(gbuf, accum.at[dixv.at[j]], add=True)
        return 0

    lax.fori_loop(0, KMAX, chunk, 0)
    pltpu.sync_copy(accum.at[pl.ds(s * BN, BN)],
                    out_hbm.at[pl.ds(w * BN, BN)])


_BM = 512
_SCALE = 1.0 / math.sqrt(1.0 + EPS)


def _mm_body(x_ref, w_ref, g_ref, b_ref, o_ref):
    h = jnp.maximum(x_ref[...] * (g_ref[...] * _SCALE) + b_ref[...], 0.0)
    o_ref[...] = jnp.dot(h, w_ref[...], preferred_element_type=jnp.float32)


def _bn_relu_mm(x, wcat, gamma, beta):
    m = x.shape[0]
    grid = (m + _BM - 1) // _BM
    return pl.pallas_call(
        _mm_body,
        grid=(grid,),
        in_specs=[
            pl.BlockSpec((_BM, C), lambda i: (i, 0)),
            pl.BlockSpec((C, K27 * C), lambda i: (0, 0)),
            pl.BlockSpec((1, C), lambda i: (0, 0)),
            pl.BlockSpec((1, C), lambda i: (0, 0)),
        ],
        out_specs=pl.BlockSpec((_BM, K27 * C), lambda i: (i, 0)),
        out_shape=jax.ShapeDtypeStruct((m, K27 * C), jnp.float32),
    )(x, wcat, gamma.reshape(1, C), beta.reshape(1, C))


def kernel(feat, pos, training, W1, W2, gamma1, beta1, gamma2, beta2):
    del training
    feat_p = jnp.pad(feat, ((0, NP - N), (0, 0)))
    pos_f = pos.reshape(N * 3)
    grid = _grid_kernel(pos_f)
    ridx, dstl = _edge_kernel(pos_f, grid)

    w1c = jnp.transpose(W1, (1, 0, 2)).reshape(C, K27 * C)
    w2c = jnp.transpose(W2, (1, 0, 2)).reshape(C, K27 * C)

    y1 = _bn_relu_mm(feat_p, w1c, gamma1, beta1).reshape(NP * K27, C)
    h1 = _acc_kernel(y1, jnp.zeros((NP, C), jnp.float32), ridx, dstl)
    y2 = _bn_relu_mm(h1, w2c, gamma2, beta2).reshape(NP * K27, C)
    out = _acc_kernel(y2, feat_p, ridx, dstl)
    return out[:N]


# speedup vs baseline: 23.1492x; 23.1492x over previous
import functools
import math

import jax
import jax.numpy as jnp
from jax import lax
from jax.experimental import pallas as pl
from jax.experimental.pallas import tpu as pltpu
from jax.experimental.pallas import tpu_sc as plsc

N = 10000
C = 128
G = 64
G3 = G * G * G
K27 = 27
EPS = 1e-4

NSC = 2
NT = 16
NW = NSC * NT
BN = 320
NP = NW * BN
TRASH = BN
GCH = G3 // NW
NB16 = BN // 16
NSLOT = K27 * NB16
GK = (NSLOT * 16 + 127) // 128
CH = 128
SPC = CH // 16
KMAX = GK

_MESH = plsc.VectorSubcoreMesh(core_axis_name="c", subcore_axis_name="s")
_SC_PARAMS = pltpu.CompilerParams(needs_layout_passes=False)


def _iota16():
    return lax.iota(jnp.int32, 16)


@functools.partial(
    pl.kernel,
    out_type=jax.ShapeDtypeStruct((G3,), jnp.int32),
    compiler_params=_SC_PARAMS,
    mesh=_MESH,
    scratch_types=[
        pltpu.VMEM((N * 3,), jnp.float32),
        pltpu.VMEM((GCH,), jnp.int32),
    ],
)
def _grid_kernel(pos_hbm, grid_hbm, posv, gchunk):
    c = lax.axis_index("c")
    s = lax.axis_index("s")
    w = c * NT + s
    base = w * GCH
    pltpu.sync_copy(pos_hbm, posv)

    def fill(i, _):
        gchunk[pl.ds(i * 16, 16)] = jnp.full((16,), -1, jnp.int32)
        return 0

    lax.fori_loop(0, GCH // 16, fill, 0)

    def scan(i, _):
        n = i * 16 + _iota16()
        f = n * 3
        xi = plsc.load_gather(posv, [f]).astype(jnp.int32)
        yi = plsc.load_gather(posv, [f + 1]).astype(jnp.int32)
        zi = plsc.load_gather(posv, [f + 2]).astype(jnp.int32)
        key = xi * (G * G) + yi * G + zi
        m = (key >= base) & (key < base + GCH)
        plsc.store_scatter(gchunk, [key - base], n, mask=m)
        return 0

    lax.fori_loop(0, N // 16, scan, 0)
    pltpu.sync_copy(gchunk, grid_hbm.at[pl.ds(base, GCH)])


@functools.partial(
    pl.kernel,
    out_type=(
        jax.ShapeDtypeStruct((NW, KMAX, CH), jnp.int32),
        jax.ShapeDtypeStruct((NW, KMAX, CH), jnp.int32),
        jax.ShapeDtypeStruct((NW, 16), jnp.int32),
    ),
    compiler_params=_SC_PARAMS,
    mesh=_MESH,
    scratch_types=[
        pltpu.VMEM((N * 3,), jnp.float32),
        pltpu.VMEM((GK, 128), jnp.int32),
        pltpu.VMEM((GK, 128), jnp.int32),
        pltpu.VMEM((GK, 128), jnp.int32),
        pltpu.VMEM((KMAX, CH), jnp.int32),
        pltpu.VMEM((KMAX, CH), jnp.int32),
        pltpu.VMEM((16,), jnp.int32),
        pltpu.SemaphoreType.DMA,
    ],
)
def _edge_kernel(pos_hbm, grid_hbm, ridx_hbm, dstl_hbm, cnt_hbm,
                 posv, nkv, inbv, gv, rv, dv, cv, sem):
    c = lax.axis_index("c")
    s = lax.axis_index("s")
    w = c * NT + s
    base_n = w * BN
    pltpu.sync_copy(pos_hbm, posv)

    def ibody(i, _):
        ln = i * 16 + _iota16()
        n = base_n + ln
        f = jnp.minimum(n, N - 1) * 3
        xi = plsc.load_gather(posv, [f]).astype(jnp.int32)
        yi = plsc.load_gather(posv, [f + 1]).astype(jnp.int32)
        zi = plsc.load_gather(posv, [f + 2]).astype(jnp.int32)
        real = n < N

        def kbody(k, _):
            dx = k // 9 - 1
            dy = (k // 3) % 3 - 1
            dz = k % 3 - 1
            nx = xi + dx
            ny = yi + dy
            nz = zi + dz
            inb = ((nx | ny | nz).astype(jnp.uint32) < G) & real
            nkey = nx * (G * G) + ny * G + nz
            j = k * NB16 + i
            row = j // 8
            col = (j % 8) * 16
            nkv[row, pl.ds(col, 16)] = jnp.where(inb, nkey, 0)
            inbv[row, pl.ds(col, 16)] = inb.astype(jnp.int32)
            return 0

        lax.fori_loop(0, K27, kbody, 0)
        return 0

    lax.fori_loop(0, NB16, ibody, 0)

    def fire(j, _):
        pltpu.async_copy(grid_hbm.at[nkv.at[j]], gv.at[j], sem)
        return 0

    lax.fori_loop(0, GK, fire, 0)
    pltpu.make_async_copy(ridx_hbm.at[w], rv, sem).wait()

    def prefill(j, _):
        row = j // SPC
        col = (j % SPC) * 16
        rv[row, pl.ds(col, 16)] = jnp.zeros((16,), jnp.int32)
        dv[row, pl.ds(col, 16)] = jnp.full((16,), TRASH, jnp.int32)
        return 0

    lax.fori_loop(0, KMAX * SPC, prefill, 0)

    def kbody2(k, ptr):
        def ibody2(i, ptr):
            j = k * NB16 + i
            row = j // 8
            col = (j % 8) * 16
            g = gv[row, pl.ds(col, 16)]
            inb = inbv[row, pl.ds(col, 16)]
            valid = (inb > 0) & (g >= 0)
            ln = i * 16 + _iota16()
            f = ptr * 16 + _iota16()
            plsc.store_scatter(rv, [f >> 7, f & (CH - 1)],
                               k * NP + g, mask=valid)
            plsc.store_scatter(dv, [f >> 7, f & (CH - 1)], ln, mask=valid)
            return ptr + valid.astype(jnp.int32)

        return lax.fori_loop(0, NB16, ibody2, ptr)

    ptr = lax.fori_loop(0, K27, kbody2, jnp.zeros((16,), jnp.int32))
    cv[...] = ptr
    pltpu.sync_copy(rv, ridx_hbm.at[w])
    pltpu.sync_copy(dv, dstl_hbm.at[w])
    pltpu.sync_copy(cv, cnt_hbm.at[w])


@functools.partial(
    pl.kernel,
    out_type=jax.ShapeDtypeStruct((NP, C), jnp.float32),
    compiler_params=_SC_PARAMS,
    mesh=_MESH,
    scratch_types=[
        pltpu.VMEM((KMAX, CH), jnp.int32),
        pltpu.VMEM((KMAX, CH), jnp.int32),
        pltpu.VMEM((16,), jnp.int32),
        pltpu.VMEM((CH, C), jnp.float32),
        pltpu.VMEM((BN + 1, C), jnp.float32),
        pltpu.SemaphoreType.DMA,
    ],
)
def _acc_kernel(y_hbm, init_hbm, ridx_hbm, dstl_hbm, cnt_hbm, out_hbm,
                rixv, dixv, cv, gbuf, acc, sem):
    c = lax.axis_index("c")
    s = lax.axis_index("s")
    w = c * NT + s
    pltpu.sync_copy(ridx_hbm.at[w], rixv)
    pltpu.sync_copy(dstl_hbm.at[w], dixv)
    pltpu.sync_copy(cnt_hbm.at[w], cv)
    pltpu.sync_copy(init_hbm.at[pl.ds(w * BN, BN)], acc.at[pl.ds(0, BN)])
    nslots = jnp.max(cv[...])
    nch = (nslots + SPC - 1) // SPC

    lanes = [jj * 16 + _iota16() for jj in range(SPC)]

    def chunk(t, _):
        pltpu.async_copy(y_hbm.at[rixv.at[t]], gbuf, sem).wait()
        dvecs = [dixv[t, pl.ds(jj * 16, 16)] for jj in range(SPC)]

        def cbody(ch, _):
            cc = jnp.zeros((16,), jnp.int32) + ch
            vals = [plsc.load_gather(gbuf, [lanes[jj], cc])
                    for jj in range(SPC)]
            for jj in range(SPC):
                plsc.addupdate_scatter(acc, [dvecs[jj], cc], vals[jj])
            return 0

        lax.fori_loop(0, C, cbody, 0)
        return 0

    lax.fori_loop(0, nch, chunk, 0)
    pltpu.sync_copy(acc.at[pl.ds(0, BN)], out_hbm.at[pl.ds(w * BN, BN)])


_BM = 512
_SCALE = 1.0 / math.sqrt(1.0 + EPS)


def _mm_body(x_ref, w_ref, g_ref, b_ref, o_ref):
    h = jnp.maximum(x_ref[...] * (g_ref[...] * _SCALE) + b_ref[...], 0.0)
    y = jnp.dot(h, w_ref[...], preferred_element_type=jnp.float32)
    for k in range(K27):
        o_ref[k] = y[:, k * C:(k + 1) * C]


def _bn_relu_mm(x, wcat, gamma, beta):
    m = x.shape[0]
    grid = (m + _BM - 1) // _BM
    return pl.pallas_call(
        _mm_body,
        grid=(grid,),
        in_specs=[
            pl.BlockSpec((_BM, C), lambda i: (i, 0)),
            pl.BlockSpec((C, K27 * C), lambda i: (0, 0)),
            pl.BlockSpec((1, C), lambda i: (0, 0)),
            pl.BlockSpec((1, C), lambda i: (0, 0)),
        ],
        out_specs=pl.BlockSpec((K27, _BM, C), lambda i: (0, i, 0)),
        out_shape=jax.ShapeDtypeStruct((K27, m, C), jnp.float32),
    )(x, wcat, gamma.reshape(1, C), beta.reshape(1, C))


def kernel(feat, pos, training, W1, W2, gamma1, beta1, gamma2, beta2):
    del training
    feat_p = jnp.pad(feat, ((0, NP - N), (0, 0)))
    pos_f = pos.reshape(N * 3)
    grid = _grid_kernel(pos_f)
    ridx, dstl, cnt = _edge_kernel(pos_f, grid)

    w1c = jnp.transpose(W1, (1, 0, 2)).reshape(C, K27 * C)
    w2c = jnp.transpose(W2, (1, 0, 2)).reshape(C, K27 * C)

    y1 = _bn_relu_mm(feat_p, w1c, gamma1, beta1).reshape(K27 * NP, C)
    h1 = _acc_kernel(y1, jnp.zeros((NP, C), jnp.float32), ridx, dstl, cnt)
    y2 = _bn_relu_mm(h1, w2c, gamma2, beta2).reshape(K27 * NP, C)
    out = _acc_kernel(y2, feat_p, ridx, dstl, cnt)
    return out[:N]
